# Initial kernel scaffold; baseline (speedup 1.0000x reference)
import jax
import jax.numpy as jnp
from jax.experimental import pallas as pl


def kernel(x, codebooks):
    raise NotImplementedError("write your pallas kernel here")


# speedup vs baseline: 3.0104x; 3.0104x over previous
import jax
import jax.numpy as jnp
from jax import lax
from jax.experimental import pallas as pl
from jax.experimental.pallas import tpu as pltpu

_NSUBQ = 4
_K = 256
_D = 64
_W = 64
_P = 196
_N = _W * _P
_PC = 49
_CHUNK = _PC * _W
_NCHUNK = _N // _CHUNK


def _pq_body(cb2_ref, cbt_ref, b2_ref, x_ref, out_ref):
    cb2 = cb2_ref[0]
    cbt = cbt_ref[0]
    b2 = b2_ref[0]
    for c in range(_NCHUNK):
        xc = x_ref[0, 0, :, c * _CHUNK:(c + 1) * _CHUNK]
        s = lax.dot(cb2, xc, preferred_element_type=jnp.float32,
                    precision=lax.Precision.DEFAULT)
        s = s + b2
        m = jnp.min(s, axis=0, keepdims=True)
        iota = lax.broadcasted_iota(jnp.int32, s.shape, 0)
        idx = jnp.min(jnp.where(s <= m, iota, jnp.int32(1 << 30)),
                      axis=0, keepdims=True)
        oh = (iota == idx).astype(jnp.float32)
        q = lax.dot(cbt, oh, preferred_element_type=jnp.float32,
                    precision=lax.Precision.HIGHEST)
        qt = jnp.swapaxes(q.reshape(_D, _PC, _W), 1, 2)
        out_ref[0, :, :, c * _PC:(c + 1) * _PC] = qt


def kernel(x, codebooks):
    batch = x.shape[0]
    xr = x.reshape(batch, _NSUBQ, _D, _N)
    cb2 = -2.0 * codebooks
    cbt = jnp.swapaxes(codebooks, 1, 2)
    b2 = jnp.sum(codebooks * codebooks, axis=2)[:, :, None]
    return pl.pallas_call(
        _pq_body,
        grid=(batch, _NSUBQ),
        in_specs=[
            pl.BlockSpec((1, _K, _D), lambda b, i: (i, 0, 0)),
            pl.BlockSpec((1, _D, _K), lambda b, i: (i, 0, 0)),
            pl.BlockSpec((1, _K, 1), lambda b, i: (i, 0, 0)),
            pl.BlockSpec((1, 1, _D, _N), lambda b, i: (b, i, 0, 0)),
        ],
        out_specs=pl.BlockSpec((1, _D, _W, _P), lambda b, i: (b, i, 0, 0)),
        out_shape=jax.ShapeDtypeStruct((batch, _NSUBQ * _D, _W, _P),
                                       jnp.float32),
        compiler_params=pltpu.CompilerParams(
            dimension_semantics=("parallel", "arbitrary")),
    )(cb2, cbt, b2, xr)


# speedup vs baseline: 3.5233x; 1.1704x over previous
import functools

import jax
import jax.numpy as jnp
from jax import lax
from jax.experimental import pallas as pl
from jax.experimental.pallas import tpu as pltpu
from jax.experimental.pallas import tpu_sc as plsc

_NSUBQ = 4
_K = 256
_D = 64
_W = 64
_P = 196
_N = _W * _P
_PC = 49
_CHUNK = _PC * _W
_NCHUNK = _N // _CHUNK

_NC = 2
_NS = 16
_NW = _NC * _NS
_CPW = _K // _NW
_WH = _W // 4


def _idx_body(cb2_ref, b2_ref, x_ref, idx_ref):
    cb2 = cb2_ref[0]
    b2 = b2_ref[0]
    for c in range(_NCHUNK):
        xc = x_ref[0, 0, :, c * _CHUNK:(c + 1) * _CHUNK]
        s = lax.dot(cb2, xc, preferred_element_type=jnp.float32,
                    precision=lax.Precision.DEFAULT)
        s = s + b2
        m = jnp.min(s, axis=0, keepdims=True)
        iota = lax.broadcasted_iota(jnp.int32, s.shape, 0)
        idx = jnp.min(jnp.where(s <= m, iota, jnp.int32(1 << 30)),
                      axis=0, keepdims=True)
        idx_ref[0, 0, :, c * _CHUNK:(c + 1) * _CHUNK] = idx


def _indices(xr, cb2, b2, batch):
    return pl.pallas_call(
        _idx_body,
        grid=(batch, _NSUBQ),
        in_specs=[
            pl.BlockSpec((1, _K, _D), lambda b, i: (i, 0, 0)),
            pl.BlockSpec((1, _K, 1), lambda b, i: (i, 0, 0)),
            pl.BlockSpec((1, 1, _D, _N), lambda b, i: (b, i, 0, 0)),
        ],
        out_specs=pl.BlockSpec((1, 1, 1, _N), lambda b, i: (b, i, 0, 0)),
        out_shape=jax.ShapeDtypeStruct((batch, _NSUBQ, 1, _N), jnp.int32),
        compiler_params=pltpu.CompilerParams(
            dimension_semantics=("parallel", "arbitrary")),
    )(cb2, b2, xr)


def _sc_gather(cbt, idx_all, batch):
    mesh = plsc.VectorSubcoreMesh(core_axis_name="c", subcore_axis_name="s")

    @functools.partial(
        pl.kernel, mesh=mesh,
        out_type=jax.ShapeDtypeStruct((batch, _K, _W, _P), jnp.float32),
        compiler_params=pltpu.CompilerParams(needs_layout_passes=False),
        scratch_types=[
            pltpu.VMEM((_CPW * _K,), jnp.float32),
            pltpu.VMEM((_N,), jnp.int32),
            pltpu.VMEM((2, _CPW, _WH, _P), jnp.float32),
            pltpu.SemaphoreType.DMA,
            pltpu.SemaphoreType.DMA,
        ],
    )
    def k(cbt_hbm, idx_hbm, out_hbm, cb_v, idx_v, buf_v, sem0, sem1):
        wid = lax.axis_index("s") * _NC + lax.axis_index("c")
        sq = wid // (_NW // _NSUBQ)
        c0 = (wid % (_NW // _NSUBQ)) * _CPW
        ch0 = sq * _D + c0
        pltpu.sync_copy(cbt_hbm.at[sq, pl.ds(c0 * _K, _CPW * _K)], cb_v)
        iota64 = lax.broadcasted_iota(jnp.int32, (16,), 0) * jnp.int32(_W)

        def step(t, carry):
            b = t // 4
            half = t % 4
            bank = t % 2

            @pl.when(half == 0)
            def _():
                pltpu.sync_copy(idx_hbm.at[b, sq, 0, :], idx_v)

            dst = out_hbm.at[b, pl.ds(ch0, _CPW), pl.ds(half * _WH, _WH), :]

            @pl.when((t >= 2) & (bank == 0))
            def _():
                pltpu.make_async_copy(buf_v.at[0], dst, sem0).wait()

            @pl.when((t >= 2) & (bank == 1))
            def _():
                pltpu.make_async_copy(buf_v.at[1], dst, sem1).wait()

            bbuf = buf_v.at[bank]

            @plsc.parallel_loop(0, _WH, unroll=2)
            def inner(u):
                w = jnp.int32(half * _WH) + u
                for j in range(13):
                    off = 180 if j == 12 else j * 16
                    pos = iota64 + (jnp.int32(off * _W) + w)
                    idxv = plsc.load_gather(idx_v, [pos])
                    for c in range(_CPW):
                        vals = plsc.load_gather(
                            cb_v, [idxv + jnp.int32(c * _K)])
                        bbuf[c, u, pl.ds(off, 16)] = vals

            @pl.when(bank == 0)
            def _():
                pltpu.make_async_copy(buf_v.at[0], dst, sem0).start()

            @pl.when(bank == 1)
            def _():
                pltpu.make_async_copy(buf_v.at[1], dst, sem1).start()

            return carry

        lax.fori_loop(0, batch * 4, step, 0)
        tail = out_hbm.at[0, pl.ds(ch0, _CPW), pl.ds(0, _WH), :]
        pltpu.make_async_copy(buf_v.at[0], tail, sem0).wait()
        pltpu.make_async_copy(buf_v.at[1], tail, sem1).wait()

    return k(cbt, idx_all)


def kernel(x, codebooks):
    batch = x.shape[0]
    xr = x.reshape(batch, _NSUBQ, _D, _N)
    cb2 = -2.0 * codebooks
    b2 = jnp.sum(codebooks * codebooks, axis=2)[:, :, None]
    cbt = jnp.swapaxes(codebooks, 1, 2).reshape(_NSUBQ, _D * _K)
    idx_all = _indices(xr, cb2, b2, batch)
    return _sc_gather(cbt, idx_all, batch)
